# baseline (speedup 1.0000x reference)
import jax
import jax.numpy as jnp
from jax import lax
from jax.experimental import pallas as pl
from jax.experimental.pallas import tpu as pltpu
from jax.experimental.pallas import tpu_sc as plsc

_INFO = plsc.get_sparse_core_info()
_NC = _INFO.num_cores
_NS = _INFO.num_subcores
_NW = _NC * _NS

_B, _C, _H, _W = 4, 3, 512, 512
_PLANE = _H * _W
_PIX = _B * _PLANE
_PPW = _PIX // _NW
_CH = 2048
_NCHUNK = _PPW // _CH
_G = 128
_NG = _CH // _G
_NVEC = _CH // 16


def _sc_body(img_hbm, w_hbm, b_hbm, out_hbm,
             xbufs, ibufs, wbufs, bbufs, obufs, sem):
    wid = lax.axis_index("s") * _NC + lax.axis_index("c")
    bi = wid // 8
    po = (wid % 8) * _PPW
    base = bi * (_C * _PLANE) + po

    iota = lax.iota(jnp.int32, 16)

    def chunk_body(s, _):
        off = pl.multiple_of(base + s * _CH, 2048)
        for c in range(3):
            pltpu.sync_copy(img_hbm.at[pl.ds(off + c * _PLANE, _CH)],
                            xbufs[c])

        def idx_body(j, _):
            p = pl.ds(j * 16, 16)
            rv = xbufs[0][p]
            gv = xbufs[1][p]
            bv = xbufs[2][p]
            fidx = rv * 65536.0 + gv * 256.0 + bv
            i1 = fidx.astype(jnp.int32)
            i3 = i1 + i1 + i1
            ibufs[0][p] = i3
            ibufs[1][p] = i3 + 1
            ibufs[2][p] = i3 + 2
            return 0

        lax.fori_loop(0, _NVEC, idx_body, 0, unroll=4)

        copies = []
        for g in range(_NG):
            gs = pl.ds(g * _G, _G)
            for c in range(3):
                isl = ibufs[c].at[gs]
                copies.append(pltpu.async_copy(
                    w_hbm.at[isl], wbufs[c].at[gs], sem))
                copies.append(pltpu.async_copy(
                    b_hbm.at[isl], bbufs[c].at[gs], sem))
        for cp in copies:
            cp.wait()

        def fx_body(j, _):
            p = pl.ds(j * 16, 16)
            for c in range(3):
                wv = wbufs[c][p]
                bv = bbufs[c][p]
                xv = xbufs[c][p]
                obufs[c][p] = wv * xv + (bv - wv + 1.0) * 127.0
            return 0

        lax.fori_loop(0, _NVEC, fx_body, 0, unroll=4)

        for c in range(3):
            pltpu.sync_copy(obufs[c],
                            out_hbm.at[pl.ds(off + c * _PLANE, _CH)])
        return 0

    lax.fori_loop(0, _NCHUNK, chunk_body, 0)


@jax.jit
def kernel(img, w, b):
    img_flat = img.reshape(-1)
    w_flat = w.reshape(-1)
    b_flat = b.reshape(-1)
    mesh = plsc.VectorSubcoreMesh(core_axis_name="c", subcore_axis_name="s")
    out_flat = pl.kernel(
        _sc_body,
        out_type=jax.ShapeDtypeStruct((_B * _C * _PLANE,), jnp.float32),
        mesh=mesh,
        scratch_types=[
            [pltpu.VMEM((_CH,), jnp.float32)] * 3,
            [pltpu.VMEM((_CH,), jnp.int32)] * 3,
            [pltpu.VMEM((_CH,), jnp.float32)] * 3,
            [pltpu.VMEM((_CH,), jnp.float32)] * 3,
            [pltpu.VMEM((_CH,), jnp.float32)] * 3,
            pltpu.SemaphoreType.DMA,
        ],
    )(img_flat, w_flat, b_flat)
    return out_flat.reshape(_B, _C, _H, _W)


# speedup vs baseline: 40.5045x; 40.5045x over previous
import jax
import jax.numpy as jnp
from jax import lax
from jax.experimental import pallas as pl
from jax.experimental.pallas import tpu as pltpu
from jax.experimental.pallas import tpu_sc as plsc

_INFO = plsc.get_sparse_core_info()
_NC = _INFO.num_cores
_NS = _INFO.num_subcores
_NW = _NC * _NS

_B, _C, _H, _W = 4, 3, 512, 512
_PLANE = _H * _W
_PIX = _B * _PLANE
_PPW = _PIX // _NW
_CH = 2048
_NCHUNK = _PPW // _CH
_G = 128
_NG = _CH // _G
_NVEC = _CH // 16


def _sc_body(img_hbm, w0_hbm, w1_hbm, w2_hbm, b0_hbm, b1_hbm, b2_hbm, out_hbm,
             xbufs, idxbuf, wbufs, bbufs, obufs, sem):
    w_cols = (w0_hbm, w1_hbm, w2_hbm)
    b_cols = (b0_hbm, b1_hbm, b2_hbm)
    wid = lax.axis_index("s") * _NC + lax.axis_index("c")
    bi = wid // 8
    po = (wid % 8) * _PPW
    base = bi * (_C * _PLANE) + po

    def chunk_body(s, _):
        off = pl.multiple_of(base + s * _CH, 2048)
        for c in range(3):
            pltpu.sync_copy(img_hbm.at[pl.ds(off + c * _PLANE, _CH)],
                            xbufs[c])

        def idx_body(j, _):
            p = pl.ds(j * 16, 16)
            rv = xbufs[0][p]
            gv = xbufs[1][p]
            bv = xbufs[2][p]
            fidx = rv * 65536.0 + gv * 256.0 + bv
            idxbuf[p] = fidx.astype(jnp.int32)
            return 0

        lax.fori_loop(0, _NVEC, idx_body, 0, unroll=4)

        copies = []
        for g in range(_NG):
            gs = pl.ds(g * _G, _G)
            isl = idxbuf.at[gs]
            for c in range(3):
                copies.append(pltpu.async_copy(
                    w_cols[c].at[isl], wbufs[c].at[gs], sem))
                copies.append(pltpu.async_copy(
                    b_cols[c].at[isl], bbufs[c].at[gs], sem))
        for cp in copies:
            cp.wait()

        def fx_body(j, _):
            p = pl.ds(j * 16, 16)
            for c in range(3):
                wv = wbufs[c][p]
                bv = bbufs[c][p]
                xv = xbufs[c][p]
                obufs[c][p] = wv * xv + (bv - wv + 1.0) * 127.0
            return 0

        lax.fori_loop(0, _NVEC, fx_body, 0, unroll=4)

        for c in range(3):
            pltpu.sync_copy(obufs[c],
                            out_hbm.at[pl.ds(off + c * _PLANE, _CH)])
        return 0

    lax.fori_loop(0, _NCHUNK, chunk_body, 0)


@jax.jit
def kernel(img, w, b):
    img_flat = img.reshape(-1)
    mesh = plsc.VectorSubcoreMesh(core_axis_name="c", subcore_axis_name="s")
    out_flat = pl.kernel(
        _sc_body,
        out_type=jax.ShapeDtypeStruct((_B * _C * _PLANE,), jnp.float32),
        mesh=mesh,
        scratch_types=[
            [pltpu.VMEM((_CH,), jnp.float32)] * 3,
            pltpu.VMEM((_CH,), jnp.int32),
            [pltpu.VMEM((_CH,), jnp.float32)] * 3,
            [pltpu.VMEM((_CH,), jnp.float32)] * 3,
            [pltpu.VMEM((_CH,), jnp.float32)] * 3,
            pltpu.SemaphoreType.DMA,
        ],
    )(img_flat, w[:, 0], w[:, 1], w[:, 2], b[:, 0], b[:, 1], b[:, 2])
    return out_flat.reshape(_B, _C, _H, _W)
